# baseline (speedup 1.0000x reference)
import functools

import jax
import jax.numpy as jnp
import numpy as np
from jax import lax
from jax.experimental import pallas as pl
from jax.experimental.pallas import tpu as pltpu
from jax.experimental.pallas import tpu_sc as plsc

_PERM = np.concatenate([
    np.concatenate([np.arange(g * 32, g * 32 + 32, 2),
                    np.arange(g * 32 + 1, g * 32 + 32, 2)])
    for g in range(4)
]).astype(np.int32)

N = 10000
E = 320000
D = 128
CH = 128
SUPER = 1024
EPAD = E + 2048
PTRPAD = 10032
NPC = 5000
PWIN = 5024
AROWS = 5120
RPT = AROWS // 16
DUMP = NPC + 56


def _matT(x, w):
    return lax.dot_general(x, w, (((1,), (1,)), ((), ())),
                           preferred_element_type=jnp.float32)


def _tables_body(a_ref, v_ref, t_ref, wav_ref, wv_ref, wat_ref, wt_ref,
                 wa_ref, At_ref, Vt_ref, At2_ref, Tt_ref, aout_ref):
    a = a_ref[...]
    At_ref[...] = _matT(a, wav_ref[...]).astype(jnp.bfloat16)
    Vt_ref[...] = _matT(v_ref[...], wv_ref[...]).astype(jnp.bfloat16)
    At2_ref[...] = _matT(a, wat_ref[...]).astype(jnp.bfloat16)
    Tt_ref[...] = _matT(t_ref[...], wt_ref[...]).astype(jnp.bfloat16)
    aout_ref[...] = jnp.dot(a, wa_ref[...], preferred_element_type=jnp.float32)


def _tc_tables(a, v, t, wav, wvm, wat, wtm, wam):
    BR = 1000
    row = pl.BlockSpec((BR, D), lambda i: (i, 0))
    wsp = pl.BlockSpec((D, D), lambda i: (0, 0))
    return pl.pallas_call(
        _tables_body,
        grid=(N // BR,),
        in_specs=[row, row, row, wsp, wsp, wsp, wsp, wsp],
        out_specs=[row] * 5,
        out_shape=[jax.ShapeDtypeStruct((N, D), jnp.bfloat16)] * 4
        + [jax.ShapeDtypeStruct((N, D), jnp.float32)],
    )(a, v, t, wav, wvm, wat, wtm, wam)


ZROWS = 321024
_ZB = 512


def _z_body(a_ref, v_ref, wav_ref, wv_ref, z_ref):
    za = _matT(a_ref[...], wav_ref[...])
    zv = _matT(v_ref[...], wv_ref[...])
    z_ref[...] = (za * zv).astype(jnp.bfloat16)


def _tc_z(a_recv, v_recv, wav, wvm):
    rd = pl.BlockSpec((_ZB, D), lambda i: (jnp.minimum(i, E // _ZB - 1), 0))
    wsp = pl.BlockSpec((D, D), lambda i: (0, 0))
    return pl.pallas_call(
        _z_body,
        grid=(ZROWS // _ZB,),
        in_specs=[rd, rd, wsp, wsp],
        out_specs=pl.BlockSpec((_ZB, D), lambda i: (i, 0)),
        out_shape=jax.ShapeDtypeStruct((ZROWS, D), jnp.bfloat16),
    )(a_recv, v_recv, wav, wvm)


def _final_body(t_ref, v_ref, pt_ref, pv_ref, w1a_ref, w1b_ref, w2a_ref,
                w2b_ref, tu_ref, vu_ref):
    outt = pt_ref[0] * 0.5
    outv = pv_ref[0]
    tu_ref[...] = _matT(t_ref[...], w1a_ref[...]) + _matT(outt, w1b_ref[...])
    vu_ref[...] = _matT(v_ref[...], w2a_ref[...]) + _matT(outv, w2b_ref[...])


def _tc_final(t, v, pt, pv, w1a, w1b, w2a, w2b):
    BR = 1000
    nb = NPC // BR
    row = pl.BlockSpec((BR, D), lambda i: (i, 0))
    par = pl.BlockSpec((1, BR, D), lambda i: (i // nb, i % nb, 0))
    wsp = pl.BlockSpec((D, D), lambda i: (0, 0))
    return pl.pallas_call(
        _final_body,
        grid=(N // BR,),
        in_specs=[row, row, par, par, wsp, wsp, wsp, wsp],
        out_specs=[row, row],
        out_shape=[jax.ShapeDtypeStruct((N, D), jnp.float32)] * 2,
    )(t, v, pt, pv, w1a, w1b, w2a, w2b)


def _sc_body_common(tA, tB, ia_hbm, ib_hbm, ptr_hbm, z_hbm, out_hbm,
                    accum, ptrwin, ia_sb, ib_sb, dest, rA0, rB0, rA1, rB1,
                    zb0, zb1, rP, sem_i, sem_g, sem_z):
    c = lax.axis_index("c")
    s = lax.axis_index("s")

    zeros16 = jnp.zeros((16,), jnp.float32)

    def zrow(i, _):
        for cc in range(8):
            rP[i, pl.ds(cc * 16, 16)] = zeros16
        return 0

    lax.fori_loop(0, CH, zrow, 0)
    pltpu.sync_copy(rP, accum.at[pl.ds(s * RPT, CH)])
    pltpu.sync_copy(rP, accum.at[pl.ds(s * RPT + CH, CH)])
    pltpu.sync_copy(rP.at[pl.ds(0, RPT - 2 * CH)],
                    accum.at[pl.ds(s * RPT + 2 * CH, RPT - 2 * CH)])
    plsc.subcore_barrier()

    w0 = pl.multiple_of(c * NPC, 8)
    pltpu.sync_copy(ptr_hbm.at[pl.ds(w0, PWIN)], ptrwin)

    lo_c = ptrwin[pl.ds(0, 16)][0]
    hi_c = ptrwin[pl.ds(NPC, 16)][0]
    per_tile = (hi_c - lo_c + 15) >> 4
    base_s = lo_c + s * per_tile
    end_s = jnp.minimum(base_s + per_tile, hi_c)
    abase = pl.multiple_of((base_s >> 3) << 3, 8)
    nch = jnp.maximum((end_s - abase + CH - 1) >> 7, 0)
    nsc = (nch + 7) >> 3

    rows = ((rA0, rB0), (rA1, rB1))
    zbs = (zb0, zb1)

    def idx_off(k):
        return pl.multiple_of(abase + k * SUPER, 8)

    def start_idx(k, par):
        pltpu.make_async_copy(ia_hbm.at[pl.ds(idx_off(k), SUPER)],
                              ia_sb.at[pl.ds(par * SUPER, SUPER)],
                              sem_i).start()
        pltpu.make_async_copy(ib_hbm.at[pl.ds(idx_off(k), SUPER)],
                              ib_sb.at[pl.ds(par * SUPER, SUPER)],
                              sem_i).start()

    def wait_idx(par):
        pltpu.make_async_copy(ia_hbm.at[pl.ds(abase, SUPER)],
                              ia_sb.at[pl.ds(par * SUPER, SUPER)],
                              sem_i).wait()
        pltpu.make_async_copy(ib_hbm.at[pl.ds(abase, SUPER)],
                              ib_sb.at[pl.ds(par * SUPER, SUPER)],
                              sem_i).wait()

    def gather_refs(ch, par):
        q = (ch >> 3) & 1
        off = q * SUPER + (ch & 7) * CH
        rA, rB = rows[par]
        return (tA.at[ia_sb.at[pl.ds(off, CH)]], rA,
                tB.at[ib_sb.at[pl.ds(off, CH)]], rB)

    def start_gathers(ch, par):
        sa, da, sb, db = gather_refs(ch, par)
        pltpu.make_async_copy(sa, da, sem_g).start()
        pltpu.make_async_copy(sb, db, sem_g).start()

    def wait_gathers(ch, par):
        sa, da, sb, db = gather_refs(ch, par)
        pltpu.make_async_copy(sa, da, sem_g).wait()
        pltpu.make_async_copy(sb, db, sem_g).wait()

    def start_z(ch, par):
        pltpu.make_async_copy(z_hbm.at[pl.ds(abase + ch * CH, CH)],
                              zbs[par], sem_z).start()

    def wait_z(par):
        pltpu.make_async_copy(z_hbm.at[pl.ds(abase, CH)],
                              zbs[par], sem_z).wait()

    def chunk_step(ch, par):
        rA, rB = rows[par]
        wait_gathers(ch, par)

        nxt = ch + 1

        @pl.when(nxt < nch)
        def _():
            @pl.when((nxt & 7) == 0)
            def _():
                q = (nxt >> 3) & 1
                wait_idx(q)
                sc2 = (nxt >> 3) + 1

                @pl.when(sc2 < nsc)
                def _():
                    start_idx(sc2, 1 - q)

            start_gathers(nxt, 1 - par)
            if z_hbm is not None:
                start_z(nxt, 1 - par)

        base = abase + ch * CH
        for g in range(8):
            ev = base + g * 16 + lax.iota(jnp.int32, 16)
            lo = jnp.zeros((16,), jnp.int32)
            hi = jnp.full((16,), NPC + 1, jnp.int32)

            def bs(j, lh):
                lo_, hi_ = lh
                mid = (lo_ + hi_) >> 1
                pm = plsc.load_gather(ptrwin, [mid])
                le = pm <= ev
                return (jnp.where(le, mid + 1, lo_),
                        jnp.where(le, hi_, mid))

            lo, hi = lax.fori_loop(0, 13, bs, (lo, hi))
            seg = lo - 1
            valid = ((seg >= 0) & (seg < NPC) &
                     (ev >= base_s) & (ev < end_s))
            dest[0, pl.ds(g * 16, 16)] = jnp.where(valid, seg, DUMP)

        if z_hbm is not None:
            wait_z(par)
            zb = zbs[par]

        def prod(k, _):
            for cc in range(4):
                sl = pl.ds(cc * 16, 16)
                va = plsc.bitcast(rA[k, sl], jnp.bfloat16)
                vb = plsc.bitcast(rB[k, sl], jnp.bfloat16)
                pr = va * vb
                if z_hbm is not None:
                    pr = pr + plsc.bitcast(zb[k, sl], jnp.bfloat16)
                ev_, od_ = plsc.unpack(pr, format=plsc.PackFormat.INTERLEAVED)
                rP[k, pl.ds(cc * 32, 16)] = ev_
                rP[k, pl.ds(cc * 32 + 16, 16)] = od_
            return 0

        lax.fori_loop(0, CH, prod, 0)
        pltpu.sync_copy(rP, accum.at[dest.at[0]], add=True)

    @pl.when(nch > 0)
    def _():
        start_idx(0, 0)
        wait_idx(0)

        @pl.when(nsc > 1)
        def _():
            start_idx(1, 1)

        start_gathers(0, 0)
        if z_hbm is not None:
            start_z(0, 0)

    def pair(i, _):
        ch0 = 2 * i
        chunk_step(ch0, 0)

        @pl.when(ch0 + 1 < nch)
        def _():
            chunk_step(ch0 + 1, 1)

        return 0

    lax.fori_loop(0, (nch + 1) >> 1, pair, 0)
    plsc.subcore_barrier()
    pltpu.sync_copy(accum.at[pl.ds(s * RPT, RPT)],
                    out_hbm.at[pl.ds(c * AROWS + s * RPT, RPT)])


def _sc_segsum(tA, tB, ia, ib, ptr, z=None):
    mesh = plsc.VectorSubcoreMesh(core_axis_name="c", subcore_axis_name="s")
    scratch = [
        pltpu.VMEM_SHARED((AROWS, D), jnp.float32),
        pltpu.VMEM((PWIN,), jnp.int32),
        pltpu.VMEM((2 * SUPER,), jnp.int32),
        pltpu.VMEM((2 * SUPER,), jnp.int32),
        pltpu.VMEM((1, CH), jnp.int32),
        pltpu.VMEM((CH, D // 2), jnp.int32),
        pltpu.VMEM((CH, D // 2), jnp.int32),
        pltpu.VMEM((CH, D // 2), jnp.int32),
        pltpu.VMEM((CH, D // 2), jnp.int32),
        pltpu.VMEM((CH, D // 2), jnp.int32),
        pltpu.VMEM((CH, D // 2), jnp.int32),
        pltpu.VMEM((CH, D), jnp.float32),
        pltpu.SemaphoreType.DMA,
        pltpu.SemaphoreType.DMA,
        pltpu.SemaphoreType.DMA,
    ]
    if z is None:
        def body(tA_, tB_, ia_, ib_, ptr_, out_, *scr):
            _sc_body_common(tA_, tB_, ia_, ib_, ptr_, None, out_, *scr)
        args = (tA, tB, ia, ib, ptr)
    else:
        def body(tA_, tB_, ia_, ib_, ptr_, z_, out_, *scr):
            _sc_body_common(tA_, tB_, ia_, ib_, ptr_, z_, out_, *scr)
        args = (tA, tB, ia, ib, ptr, z)
    kfn = pl.kernel(
        body, mesh=mesh,
        out_type=jax.ShapeDtypeStruct((2 * AROWS, D), jnp.float32),
        scratch_types=scratch,
        compiler_params=pltpu.CompilerParams(needs_layout_passes=False,
                                             use_tc_tiling_on_sc=False),
    )
    return kfn(*args)


def kernel(t_embed, v_embed, a_embed, a_recv, v_recv, ptr_t, a_list_t,
           v_list_t, ptr_v, a_list_v, t_list_v, wv, wt, wa_v, wa_t,
           w1, w2, wa):
    i32 = jnp.int32
    pad_i = jnp.zeros((EPAD - E,), i32)
    alt = jnp.concatenate([a_list_t.astype(i32), pad_i])
    vlt = jnp.concatenate([v_list_t.astype(i32), pad_i])
    alv = jnp.concatenate([a_list_v.astype(i32), pad_i])
    tlv = jnp.concatenate([t_list_v.astype(i32), pad_i])
    pad_p = jnp.full((PTRPAD - N - 1,), E, i32)
    ptr_t_p = jnp.concatenate([ptr_t.astype(i32), pad_p])
    ptr_v_p = jnp.concatenate([ptr_v.astype(i32), pad_p])

    At, Vt, At2, Tt, a_out = _tc_tables(a_embed, v_embed, t_embed,
                                        wa_v, wv, wa_t, wt, wa)

    def _pack32(x):
        return jax.lax.bitcast_convert_type(
            x.reshape(x.shape[0], x.shape[1] // 2, 2), jnp.int32)

    pv = _sc_segsum(_pack32(At2), _pack32(Tt), alv, tlv,
                    ptr_v_p).reshape(2, AROWS, D)
    Z = _tc_z(a_recv, v_recv, wa_v, wv)
    pt = _sc_segsum(_pack32(At), _pack32(Vt), alt, vlt, ptr_t_p,
                    _pack32(Z)).reshape(2, AROWS, D)

    t_up, v_up = _tc_final(t_embed, v_embed, pt, pv,
                           w1[:, :D], w1[:, D:][:, _PERM],
                           w2[:, :D], w2[:, D:][:, _PERM])
    return (t_up, v_up, a_out)


# speedup vs baseline: 2.4561x; 2.4561x over previous
import functools

import jax
import jax.numpy as jnp
from jax import lax
from jax.experimental import pallas as pl
from jax.experimental.pallas import tpu as pltpu
from jax.experimental.pallas import tpu_sc as plsc

N = 10000
E = 320000
D = 128
CH = 128
SUPER = 1024
EPAD = E + 2048
PTRPAD = 10032
NPC = 5000
PWIN = 5024
AROWS = 5120
RPT = AROWS // 16
DUMP = NPC + 56


def _matT(x, w):
    return lax.dot_general(x, w, (((1,), (1,)), ((), ())),
                           preferred_element_type=jnp.float32)


def _tables_body(a_ref, v_ref, t_ref, wav_ref, wv_ref, wat_ref, wt_ref,
                 wa_ref, At_ref, Vt_ref, At2_ref, Tt_ref, aout_ref):
    a = a_ref[...]
    At_ref[...] = _matT(a, wav_ref[...])
    Vt_ref[...] = _matT(v_ref[...], wv_ref[...])
    At2_ref[...] = _matT(a, wat_ref[...])
    Tt_ref[...] = _matT(t_ref[...], wt_ref[...])
    aout_ref[...] = jnp.dot(a, wa_ref[...], preferred_element_type=jnp.float32)


def _tc_tables(a, v, t, wav, wvm, wat, wtm, wam):
    BR = 1000
    row = pl.BlockSpec((BR, D), lambda i: (i, 0))
    wsp = pl.BlockSpec((D, D), lambda i: (0, 0))
    return pl.pallas_call(
        _tables_body,
        grid=(N // BR,),
        in_specs=[row, row, row, wsp, wsp, wsp, wsp, wsp],
        out_specs=[row] * 5,
        out_shape=[jax.ShapeDtypeStruct((N, D), jnp.float32)] * 5,
    )(a, v, t, wav, wvm, wat, wtm, wam)


ZROWS = 321024
_ZB = 512


def _z_body(a_ref, v_ref, wav_ref, wv_ref, z_ref):
    za = _matT(a_ref[...], wav_ref[...])
    zv = _matT(v_ref[...], wv_ref[...])
    z_ref[...] = za * zv


def _tc_z(a_recv, v_recv, wav, wvm):
    rd = pl.BlockSpec((_ZB, D), lambda i: (jnp.minimum(i, E // _ZB - 1), 0))
    wsp = pl.BlockSpec((D, D), lambda i: (0, 0))
    return pl.pallas_call(
        _z_body,
        grid=(ZROWS // _ZB,),
        in_specs=[rd, rd, wsp, wsp],
        out_specs=pl.BlockSpec((_ZB, D), lambda i: (i, 0)),
        out_shape=jax.ShapeDtypeStruct((ZROWS, D), jnp.float32),
    )(a_recv, v_recv, wav, wvm)


def _final_body(t_ref, v_ref, pt_ref, pv_ref, w1_ref, w2_ref, tu_ref, vu_ref):
    outt = pt_ref[0] * 0.5
    outv = pv_ref[0]
    w1 = w1_ref[...]
    w2 = w2_ref[...]
    tu_ref[...] = _matT(t_ref[...], w1[:, :D]) + _matT(outt, w1[:, D:])
    vu_ref[...] = _matT(v_ref[...], w2[:, :D]) + _matT(outv, w2[:, D:])


def _tc_final(t, v, pt, pv, w1, w2):
    BR = 1000
    nb = NPC // BR
    row = pl.BlockSpec((BR, D), lambda i: (i, 0))
    par = pl.BlockSpec((1, BR, D), lambda i: (i // nb, i % nb, 0))
    wsp = pl.BlockSpec((D, 2 * D), lambda i: (0, 0))
    return pl.pallas_call(
        _final_body,
        grid=(N // BR,),
        in_specs=[row, row, par, par, wsp, wsp],
        out_specs=[row, row],
        out_shape=[jax.ShapeDtypeStruct((N, D), jnp.float32)] * 2,
    )(t, v, pt, pv, w1, w2)


def _sc_body_common(tA, tB, ia_hbm, ib_hbm, ptr_hbm, z_hbm, out_hbm,
                    accum, ptrwin, ia_sb, ib_sb, dest, rA0, rB0, rA1, rB1,
                    sem_i, sem_g, sem_z, sem_s):
    c = lax.axis_index("c")
    s = lax.axis_index("s")

    zeros16 = jnp.zeros((16,), jnp.float32)

    def zrow(i, _):
        for cc in range(8):
            rA0[i, pl.ds(cc * 16, 16)] = zeros16
        return 0

    lax.fori_loop(0, CH, zrow, 0)
    pltpu.sync_copy(rA0, accum.at[pl.ds(s * RPT, CH)])
    pltpu.sync_copy(rA0, accum.at[pl.ds(s * RPT + CH, CH)])
    pltpu.sync_copy(rA0.at[pl.ds(0, RPT - 2 * CH)],
                    accum.at[pl.ds(s * RPT + 2 * CH, RPT - 2 * CH)])
    plsc.subcore_barrier()

    w0 = pl.multiple_of(c * NPC, 8)
    pltpu.sync_copy(ptr_hbm.at[pl.ds(w0, PWIN)], ptrwin)

    lo_c = ptrwin[pl.ds(0, 16)][0]
    hi_c = ptrwin[pl.ds(NPC, 16)][0]
    per_tile = (hi_c - lo_c + 15) >> 4
    base_s = lo_c + s * per_tile
    end_s = jnp.minimum(base_s + per_tile, hi_c)
    abase = pl.multiple_of((base_s >> 3) << 3, 8)
    nch = jnp.maximum((end_s - abase + CH - 1) >> 7, 0)
    nsc = (nch + 7) >> 3

    rows = ((rA0, rB0), (rA1, rB1))

    def idx_off(k):
        return pl.multiple_of(abase + k * SUPER, 8)

    def start_idx(k, par):
        pltpu.make_async_copy(ia_hbm.at[pl.ds(idx_off(k), SUPER)],
                              ia_sb.at[pl.ds(par * SUPER, SUPER)],
                              sem_i).start()
        pltpu.make_async_copy(ib_hbm.at[pl.ds(idx_off(k), SUPER)],
                              ib_sb.at[pl.ds(par * SUPER, SUPER)],
                              sem_i).start()

    def wait_idx(par):
        pltpu.make_async_copy(ia_hbm.at[pl.ds(abase, SUPER)],
                              ia_sb.at[pl.ds(par * SUPER, SUPER)],
                              sem_i).wait()
        pltpu.make_async_copy(ib_hbm.at[pl.ds(abase, SUPER)],
                              ib_sb.at[pl.ds(par * SUPER, SUPER)],
                              sem_i).wait()

    def gather_refs(ch, par):
        q = (ch >> 3) & 1
        off = q * SUPER + (ch & 7) * CH
        rA, rB = rows[par]
        return (tA.at[ia_sb.at[pl.ds(off, CH)]], rA,
                tB.at[ib_sb.at[pl.ds(off, CH)]], rB)

    def start_gathers(ch, par):
        sa, da, sb, db = gather_refs(ch, par)
        pltpu.make_async_copy(sa, da, sem_g).start()
        pltpu.make_async_copy(sb, db, sem_g).start()

    def wait_gathers(ch, par):
        sa, da, sb, db = gather_refs(ch, par)
        pltpu.make_async_copy(sa, da, sem_g).wait()
        pltpu.make_async_copy(sb, db, sem_g).wait()

    def wait_scatters(par):
        rA, rB = rows[par]
        pltpu.make_async_copy(rA, accum.at[dest.at[par]], sem_s).wait()
        if z_hbm is not None:
            pltpu.make_async_copy(rB, accum.at[dest.at[par]], sem_s).wait()

    def chunk_step(ch, par):
        rA, rB = rows[par]
        wait_gathers(ch, par)

        @pl.when(ch >= 1)
        def _():
            wait_scatters(1 - par)

        nxt = ch + 1

        @pl.when(nxt < nch)
        def _():
            @pl.when((nxt & 7) == 0)
            def _():
                q = (nxt >> 3) & 1
                wait_idx(q)
                sc2 = (nxt >> 3) + 1

                @pl.when(sc2 < nsc)
                def _():
                    start_idx(sc2, 1 - q)

            start_gathers(nxt, 1 - par)

        base = abase + ch * CH
        for g in range(8):
            ev = base + g * 16 + lax.iota(jnp.int32, 16)
            lo = jnp.zeros((16,), jnp.int32)
            hi = jnp.full((16,), NPC + 1, jnp.int32)

            def bs(j, lh):
                lo_, hi_ = lh
                mid = (lo_ + hi_) >> 1
                pm = plsc.load_gather(ptrwin, [mid])
                le = pm <= ev
                return (jnp.where(le, mid + 1, lo_),
                        jnp.where(le, hi_, mid))

            lo, hi = lax.fori_loop(0, 13, bs, (lo, hi))
            seg = lo - 1
            valid = ((seg >= 0) & (seg < NPC) &
                     (ev >= base_s) & (ev < end_s))
            dest[par, pl.ds(g * 16, 16)] = jnp.where(valid, seg, DUMP)

        def prod(k, _):
            for kk in range(2):
                for cc in range(8):
                    sl = pl.ds(cc * 16, 16)
                    rA[2 * k + kk, sl] = rA[2 * k + kk, sl] * rB[2 * k + kk, sl]
            return 0

        lax.fori_loop(0, CH // 2, prod, 0)

        if z_hbm is not None:
            cpZ = pltpu.async_copy(z_hbm.at[pl.ds(base, CH)], rB, sem_z)
            pltpu.async_copy(rA, accum.at[dest.at[par]], sem_s, add=True)
            cpZ.wait()
            pltpu.async_copy(rB, accum.at[dest.at[par]], sem_s, add=True)
        else:
            pltpu.async_copy(rA, accum.at[dest.at[par]], sem_s, add=True)

    @pl.when(nch > 0)
    def _():
        start_idx(0, 0)
        wait_idx(0)

        @pl.when(nsc > 1)
        def _():
            start_idx(1, 1)

        start_gathers(0, 0)

    def pair(i, _):
        ch0 = 2 * i
        chunk_step(ch0, 0)

        @pl.when(ch0 + 1 < nch)
        def _():
            chunk_step(ch0 + 1, 1)

        return 0

    lax.fori_loop(0, (nch + 1) >> 1, pair, 0)

    lastpar = (nch - 1) & 1

    @pl.when((nch > 0) & (lastpar == 0))
    def _():
        wait_scatters(0)

    @pl.when((nch > 0) & (lastpar == 1))
    def _():
        wait_scatters(1)

    plsc.subcore_barrier()
    pltpu.sync_copy(accum.at[pl.ds(s * RPT, RPT)],
                    out_hbm.at[pl.ds(c * AROWS + s * RPT, RPT)])


def _sc_segsum(tA, tB, ia, ib, ptr, z=None):
    mesh = plsc.VectorSubcoreMesh(core_axis_name="c", subcore_axis_name="s")
    scratch = [
        pltpu.VMEM_SHARED((AROWS, D), jnp.float32),
        pltpu.VMEM((PWIN,), jnp.int32),
        pltpu.VMEM((2 * SUPER,), jnp.int32),
        pltpu.VMEM((2 * SUPER,), jnp.int32),
        pltpu.VMEM((2, CH), jnp.int32),
        pltpu.VMEM((CH, D), jnp.float32),
        pltpu.VMEM((CH, D), jnp.float32),
        pltpu.VMEM((CH, D), jnp.float32),
        pltpu.VMEM((CH, D), jnp.float32),
        pltpu.SemaphoreType.DMA,
        pltpu.SemaphoreType.DMA,
        pltpu.SemaphoreType.DMA,
        pltpu.SemaphoreType.DMA,
    ]
    if z is None:
        def body(tA_, tB_, ia_, ib_, ptr_, out_, *scr):
            _sc_body_common(tA_, tB_, ia_, ib_, ptr_, None, out_, *scr)
        args = (tA, tB, ia, ib, ptr)
    else:
        def body(tA_, tB_, ia_, ib_, ptr_, z_, out_, *scr):
            _sc_body_common(tA_, tB_, ia_, ib_, ptr_, z_, out_, *scr)
        args = (tA, tB, ia, ib, ptr, z)
    kfn = pl.kernel(
        body, mesh=mesh,
        out_type=jax.ShapeDtypeStruct((2 * AROWS, D), jnp.float32),
        scratch_types=scratch,
        compiler_params=pltpu.CompilerParams(needs_layout_passes=False),
    )
    return kfn(*args)


def kernel(t_embed, v_embed, a_embed, a_recv, v_recv, ptr_t, a_list_t,
           v_list_t, ptr_v, a_list_v, t_list_v, wv, wt, wa_v, wa_t,
           w1, w2, wa):
    i32 = jnp.int32
    pad_i = jnp.zeros((EPAD - E,), i32)
    alt = jnp.concatenate([a_list_t.astype(i32), pad_i])
    vlt = jnp.concatenate([v_list_t.astype(i32), pad_i])
    alv = jnp.concatenate([a_list_v.astype(i32), pad_i])
    tlv = jnp.concatenate([t_list_v.astype(i32), pad_i])
    pad_p = jnp.full((PTRPAD - N - 1,), E, i32)
    ptr_t_p = jnp.concatenate([ptr_t.astype(i32), pad_p])
    ptr_v_p = jnp.concatenate([ptr_v.astype(i32), pad_p])

    At, Vt, At2, Tt, a_out = _tc_tables(a_embed, v_embed, t_embed,
                                        wa_v, wv, wa_t, wt, wa)
    pv = _sc_segsum(At2, Tt, alv, tlv, ptr_v_p).reshape(2, AROWS, D)
    Z = _tc_z(a_recv, v_recv, wa_v, wv)
    pt = _sc_segsum(At, Vt, alt, vlt, ptr_t_p, Z).reshape(2, AROWS, D)

    t_up, v_up = _tc_final(t_embed, v_embed, pt, pv, w1, w2)
    return (t_up, v_up, a_out)


# speedup vs baseline: 2.4581x; 1.0008x over previous
import jax
import jax.numpy as jnp
from jax import lax
from jax.experimental import pallas as pl
from jax.experimental.pallas import tpu as pltpu
from jax.experimental.pallas import tpu_sc as plsc

N = 10000
E = 320000
D = 128
CH = 128
SUPER = 1024
EPAD = E + 2048
PTRPAD = 10032
NPC = 5000
PWIN = 5024
AROWS = 5120
RPT = AROWS // 16
DUMP = NPC + 56


def _matT(x, w):
    return lax.dot_general(x, w, (((1,), (1,)), ((), ())),
                           preferred_element_type=jnp.float32)


def _tables_body(a_ref, v_ref, t_ref, wav_ref, wv_ref, wat_ref, wt_ref,
                 wa_ref, At_ref, Vt_ref, At2_ref, Tt_ref, aout_ref):
    a = a_ref[...]
    At_ref[...] = _matT(a, wav_ref[...])
    Vt_ref[...] = _matT(v_ref[...], wv_ref[...])
    At2_ref[...] = _matT(a, wat_ref[...])
    Tt_ref[...] = _matT(t_ref[...], wt_ref[...])
    aout_ref[...] = jnp.dot(a, wa_ref[...], preferred_element_type=jnp.float32)


def _tc_tables(a, v, t, wav, wvm, wat, wtm, wam):
    BR = 1000
    row = pl.BlockSpec((BR, D), lambda i: (i, 0))
    wsp = pl.BlockSpec((D, D), lambda i: (0, 0))
    return pl.pallas_call(
        _tables_body,
        grid=(N // BR,),
        in_specs=[row, row, row, wsp, wsp, wsp, wsp, wsp],
        out_specs=[row] * 5,
        out_shape=[jax.ShapeDtypeStruct((N, D), jnp.float32)] * 5,
    )(a, v, t, wav, wvm, wat, wtm, wam)


ZROWS = 321024
_ZB = 512


def _z_body(a_ref, v_ref, wav_ref, wv_ref, z_ref):
    za = _matT(a_ref[...], wav_ref[...])
    zv = _matT(v_ref[...], wv_ref[...])
    z_ref[...] = za * zv


def _tc_z(a_recv, v_recv, wav, wvm):
    rd = pl.BlockSpec((_ZB, D), lambda i: (jnp.minimum(i, E // _ZB - 1), 0))
    wsp = pl.BlockSpec((D, D), lambda i: (0, 0))
    return pl.pallas_call(
        _z_body,
        grid=(ZROWS // _ZB,),
        in_specs=[rd, rd, wsp, wsp],
        out_specs=pl.BlockSpec((_ZB, D), lambda i: (i, 0)),
        out_shape=jax.ShapeDtypeStruct((ZROWS, D), jnp.float32),
    )(a_recv, v_recv, wav, wvm)


def _final_body(t_ref, v_ref, pt_ref, pv_ref, w1_ref, w2_ref, tu_ref, vu_ref):
    outt = pt_ref[0] * 0.5
    outv = pv_ref[0]
    w1 = w1_ref[...]
    w2 = w2_ref[...]
    tu_ref[...] = _matT(t_ref[...], w1[:, :D]) + _matT(outt, w1[:, D:])
    vu_ref[...] = _matT(v_ref[...], w2[:, :D]) + _matT(outv, w2[:, D:])


def _tc_final(t, v, pt, pv, w1, w2):
    BR = 1000
    nb = NPC // BR
    row = pl.BlockSpec((BR, D), lambda i: (i, 0))
    par = pl.BlockSpec((1, BR, D), lambda i: (i // nb, i % nb, 0))
    wsp = pl.BlockSpec((D, 2 * D), lambda i: (0, 0))
    return pl.pallas_call(
        _final_body,
        grid=(N // BR,),
        in_specs=[row, row, par, par, wsp, wsp],
        out_specs=[row, row],
        out_shape=[jax.ShapeDtypeStruct((N, D), jnp.float32)] * 2,
    )(t, v, pt, pv, w1, w2)


def _sc_body_common(tA, tB, ia_hbm, ib_hbm, ptr_hbm, z_hbm, out_hbm,
                    accum, ptrwin, ia_sb, ib_sb, dest, rA0, rB0, rA1, rB1,
                    sem_i, sem_g, sem_z, sem_s):
    c = lax.axis_index("c")
    s = lax.axis_index("s")

    zeros16 = jnp.zeros((16,), jnp.float32)

    def zrow(i, _):
        for cc in range(8):
            rA0[i, pl.ds(cc * 16, 16)] = zeros16
        return 0

    lax.fori_loop(0, CH, zrow, 0)
    pltpu.sync_copy(rA0, accum.at[pl.ds(s * RPT, CH)])
    pltpu.sync_copy(rA0, accum.at[pl.ds(s * RPT + CH, CH)])
    pltpu.sync_copy(rA0.at[pl.ds(0, RPT - 2 * CH)],
                    accum.at[pl.ds(s * RPT + 2 * CH, RPT - 2 * CH)])
    plsc.subcore_barrier()

    w0 = pl.multiple_of(c * NPC, 8)
    pltpu.sync_copy(ptr_hbm.at[pl.ds(w0, PWIN)], ptrwin)

    lo_c = ptrwin[pl.ds(0, 16)][0]
    hi_c = ptrwin[pl.ds(NPC, 16)][0]
    per_tile = (hi_c - lo_c + 15) >> 4
    base_s = lo_c + s * per_tile
    end_s = jnp.minimum(base_s + per_tile, hi_c)
    abase = pl.multiple_of((base_s >> 3) << 3, 8)
    nch = jnp.maximum((end_s - abase + CH - 1) >> 7, 0)
    nsc = (nch + 7) >> 3

    rows = ((rA0, rB0), (rA1, rB1))

    def idx_off(k):
        return pl.multiple_of(abase + k * SUPER, 8)

    def start_idx(k, par):
        pltpu.make_async_copy(ia_hbm.at[pl.ds(idx_off(k), SUPER)],
                              ia_sb.at[pl.ds(par * SUPER, SUPER)],
                              sem_i).start()
        pltpu.make_async_copy(ib_hbm.at[pl.ds(idx_off(k), SUPER)],
                              ib_sb.at[pl.ds(par * SUPER, SUPER)],
                              sem_i).start()

    def wait_idx(par):
        pltpu.make_async_copy(ia_hbm.at[pl.ds(abase, SUPER)],
                              ia_sb.at[pl.ds(par * SUPER, SUPER)],
                              sem_i).wait()
        pltpu.make_async_copy(ib_hbm.at[pl.ds(abase, SUPER)],
                              ib_sb.at[pl.ds(par * SUPER, SUPER)],
                              sem_i).wait()

    def gather_refs(ch, par):
        q = (ch >> 3) & 1
        off = q * SUPER + (ch & 7) * CH
        rA, rB = rows[par]
        return (tA.at[ia_sb.at[pl.ds(off, CH)]], rA,
                tB.at[ib_sb.at[pl.ds(off, CH)]], rB)

    def start_gathers(ch, par):
        sa, da, sb, db = gather_refs(ch, par)
        pltpu.make_async_copy(sa, da, sem_g).start()
        pltpu.make_async_copy(sb, db, sem_g).start()

    def wait_gathers(ch, par):
        sa, da, sb, db = gather_refs(ch, par)
        pltpu.make_async_copy(sa, da, sem_g).wait()
        pltpu.make_async_copy(sb, db, sem_g).wait()

    def wait_scatters(par):
        rA, rB = rows[par]
        pltpu.make_async_copy(rA, accum.at[dest.at[par]], sem_s).wait()
        if z_hbm is not None:
            pltpu.make_async_copy(rB, accum.at[dest.at[par]], sem_s).wait()

    def chunk_step(ch, par):
        rA, rB = rows[par]
        wait_gathers(ch, par)

        @pl.when(ch >= 1)
        def _():
            wait_scatters(1 - par)

        nxt = ch + 1

        @pl.when(nxt < nch)
        def _():
            @pl.when((nxt & 7) == 0)
            def _():
                q = (nxt >> 3) & 1
                wait_idx(q)
                sc2 = (nxt >> 3) + 1

                @pl.when(sc2 < nsc)
                def _():
                    start_idx(sc2, 1 - q)

            start_gathers(nxt, 1 - par)

        base = abase + ch * CH
        for g in range(8):
            ev = base + g * 16 + lax.iota(jnp.int32, 16)
            lo = jnp.zeros((16,), jnp.int32)
            hi = jnp.full((16,), NPC + 1, jnp.int32)

            def bs(j, lh):
                lo_, hi_ = lh
                mid = (lo_ + hi_) >> 1
                pm = plsc.load_gather(ptrwin, [mid])
                le = pm <= ev
                return (jnp.where(le, mid + 1, lo_),
                        jnp.where(le, hi_, mid))

            lo, hi = lax.fori_loop(0, 13, bs, (lo, hi))
            seg = lo - 1
            valid = ((seg >= 0) & (seg < NPC) &
                     (ev >= base_s) & (ev < end_s))
            dest[par, pl.ds(g * 16, 16)] = jnp.where(valid, seg, DUMP)

        def prod(k, _):
            for kk in range(2):
                for cc in range(8):
                    sl = pl.ds(cc * 16, 16)
                    rA[2 * k + kk, sl] = rA[2 * k + kk, sl] * rB[2 * k + kk, sl]
            return 0

        lax.fori_loop(0, CH // 2, prod, 0)

        if z_hbm is not None:
            cpZ = pltpu.async_copy(z_hbm.at[pl.ds(base, CH)], rB, sem_z)
            pltpu.async_copy(rA, accum.at[dest.at[par]], sem_s, add=True)
            cpZ.wait()
            pltpu.async_copy(rB, accum.at[dest.at[par]], sem_s, add=True)
        else:
            pltpu.async_copy(rA, accum.at[dest.at[par]], sem_s, add=True)

    @pl.when(nch > 0)
    def _():
        start_idx(0, 0)
        wait_idx(0)

        @pl.when(nsc > 1)
        def _():
            start_idx(1, 1)

        start_gathers(0, 0)

    def pair(i, _):
        ch0 = 2 * i
        chunk_step(ch0, 0)

        @pl.when(ch0 + 1 < nch)
        def _():
            chunk_step(ch0 + 1, 1)

        return 0

    lax.fori_loop(0, (nch + 1) >> 1, pair, 0)

    lastpar = (nch - 1) & 1

    @pl.when((nch > 0) & (lastpar == 0))
    def _():
        wait_scatters(0)

    @pl.when((nch > 0) & (lastpar == 1))
    def _():
        wait_scatters(1)

    plsc.subcore_barrier()
    pltpu.sync_copy(accum.at[pl.ds(s * RPT, RPT)],
                    out_hbm.at[pl.ds(c * AROWS + s * RPT, RPT)])


def _sc_segsum(tA, tB, ia, ib, ptr, z=None):
    mesh = plsc.VectorSubcoreMesh(core_axis_name="c", subcore_axis_name="s")
    scratch = [
        pltpu.VMEM_SHARED((AROWS, D), jnp.float32),
        pltpu.VMEM((PWIN,), jnp.int32),
        pltpu.VMEM((2 * SUPER,), jnp.int32),
        pltpu.VMEM((2 * SUPER,), jnp.int32),
        pltpu.VMEM((2, CH), jnp.int32),
        pltpu.VMEM((CH, D), jnp.float32),
        pltpu.VMEM((CH, D), jnp.float32),
        pltpu.VMEM((CH, D), jnp.float32),
        pltpu.VMEM((CH, D), jnp.float32),
        pltpu.SemaphoreType.DMA,
        pltpu.SemaphoreType.DMA,
        pltpu.SemaphoreType.DMA,
        pltpu.SemaphoreType.DMA,
    ]
    if z is None:
        def body(tA_, tB_, ia_, ib_, ptr_, out_, *scr):
            _sc_body_common(tA_, tB_, ia_, ib_, ptr_, None, out_, *scr)
        args = (tA, tB, ia, ib, ptr)
    else:
        def body(tA_, tB_, ia_, ib_, ptr_, z_, out_, *scr):
            _sc_body_common(tA_, tB_, ia_, ib_, ptr_, z_, out_, *scr)
        args = (tA, tB, ia, ib, ptr, z)
    kfn = pl.kernel(
        body, mesh=mesh,
        out_type=jax.ShapeDtypeStruct((2 * AROWS, D), jnp.float32),
        scratch_types=scratch,
        compiler_params=pltpu.CompilerParams(needs_layout_passes=False),
    )
    return kfn(*args)


def kernel(t_embed, v_embed, a_embed, a_recv, v_recv, ptr_t, a_list_t,
           v_list_t, ptr_v, a_list_v, t_list_v, wv, wt, wa_v, wa_t,
           w1, w2, wa):
    i32 = jnp.int32
    pad_i = jnp.zeros((EPAD - E,), i32)
    alt = jnp.concatenate([a_list_t.astype(i32), pad_i])
    vlt = jnp.concatenate([v_list_t.astype(i32), pad_i])
    alv = jnp.concatenate([a_list_v.astype(i32), pad_i])
    tlv = jnp.concatenate([t_list_v.astype(i32), pad_i])
    pad_p = jnp.full((PTRPAD - N - 1,), E, i32)
    ptr_t_p = jnp.concatenate([ptr_t.astype(i32), pad_p])
    ptr_v_p = jnp.concatenate([ptr_v.astype(i32), pad_p])

    At, Vt, At2, Tt, a_out = _tc_tables(a_embed, v_embed, t_embed,
                                        wa_v, wv, wa_t, wt, wa)
    pv = _sc_segsum(At2, Tt, alv, tlv, ptr_v_p).reshape(2, AROWS, D)
    Z = _tc_z(a_recv, v_recv, wa_v, wv)
    pt = _sc_segsum(At, Vt, alt, vlt, ptr_t_p, Z).reshape(2, AROWS, D)

    t_up, v_up = _tc_final(t_embed, v_embed, pt, pv, w1, w2)
    return (t_up, v_up, a_out)
